# baseline (speedup 1.0000x reference)
import functools

import jax
import jax.numpy as jnp
from jax import lax
from jax.experimental import pallas as pl
from jax.experimental.pallas import tpu as pltpu
from jax.experimental.pallas import tpu_sc as plsc

NC = 2
NS = 16
NW = NC * NS

B = 16384
L = 20
D = 64
NVREG = D // 16
E = 2 * B
EPW = E // NW
CHUNK = 8
NCHUNK = EPW // CHUNK
UROWS = CHUNK * L
UGROUPS = UROWS // 16
PHASES = 2
CPP = NCHUNK // PHASES
UIPP = CPP * UROWS


def _sc_scores(u_idx, v_idx, u_emb, v_emb):
    mesh = plsc.VectorSubcoreMesh(
        core_axis_name="c", subcore_axis_name="s", num_cores=NC, num_subcores=NS
    )

    @functools.partial(
        pl.kernel,
        out_type=jax.ShapeDtypeStruct((E,), jnp.float32),
        mesh=mesh,
        compiler_params=pltpu.CompilerParams(needs_layout_passes=False),
        scratch_types=[
            pltpu.VMEM((UIPP,), jnp.int32),
            pltpu.VMEM((EPW + 16,), jnp.int32),
            pltpu.VMEM((2, UROWS, D), jnp.float32),
            pltpu.VMEM((2, CHUNK, D), jnp.float32),
            pltpu.VMEM((EPW,), jnp.float32),
            pltpu.SemaphoreType.DMA,
            pltpu.SemaphoreType.DMA,
            pltpu.SemaphoreType.DMA,
            pltpu.SemaphoreType.DMA,
            pltpu.SemaphoreType.DMA,
            pltpu.SemaphoreType.DMA,
            pltpu.SemaphoreType.DMA,
            pltpu.SemaphoreType.DMA,
            pltpu.SemaphoreType.DMA,
            pltpu.SemaphoreType.DMA,
        ],
    )
    def kfn(u_idx_hbm, v_idx_hbm, u_emb_hbm, v_emb_hbm, out_hbm,
            uidx_v, vidx_v, urows_v, vrows_v, score_v,
            usem0, usem1, usem2, usem3, usem4, usem5, usem6, usem7,
            vsem0, vsem1):
        wid = lax.axis_index("s") * NC + lax.axis_index("c")
        pltpu.sync_copy(v_idx_hbm.at[wid], vidx_v.at[pl.ds(0, EPW)])

        usems = ((usem0, usem1, usem2, usem3), (usem4, usem5, usem6, usem7))
        vsems = (vsem0, vsem1)
        inv_denom = 1.0 / float(L - 1)
        lane = lax.iota(jnp.int32, 16)

        for p in range(PHASES):
            pb = p * CPP
            pltpu.sync_copy(u_idx_hbm.at[wid, p], uidx_v)

            def issue(jj, slot):
                def ig(g, c):
                    iv = uidx_v[pl.ds((jj * UGROUPS + g) * 16, 16)]
                    idxs = [iv[k] for k in range(16)]
                    for k in range(16):
                        pltpu.async_copy(
                            u_emb_hbm.at[idxs[k]],
                            urows_v.at[slot, g * 16 + k],
                            usems[slot][k % 4])
                    return c
                lax.fori_loop(0, UGROUPS, ig, 0)
                ivv = vidx_v[pl.ds((pb + jj) * CHUNK, 16)]
                idxs = [ivv[k] for k in range(CHUNK)]
                for k in range(CHUNK):
                    pltpu.async_copy(
                        v_emb_hbm.at[idxs[k]],
                        vrows_v.at[slot, k],
                        vsems[slot])

            def drain(slot):
                for c4 in range(4):
                    pltpu.make_async_copy(
                        u_emb_hbm.at[pl.ds(0, UROWS // 4)],
                        urows_v.at[slot, pl.ds(c4 * (UROWS // 4), UROWS // 4)],
                        usems[slot][c4]).wait()
                pltpu.make_async_copy(
                    v_emb_hbm.at[pl.ds(0, CHUNK)], vrows_v.at[slot],
                    vsems[slot]).wait()

            def compute(jj, slot, off):
                def elem(bi, sv):
                    base = bi * L
                    accs = [urows_v[slot, base, pl.ds(k * 16, 16)]
                            for k in range(NVREG)]
                    for l in range(1, L):
                        for k in range(NVREG):
                            accs[k] = accs[k] + urows_v[slot, base + l,
                                                        pl.ds(k * 16, 16)]
                    t = accs[0] * vrows_v[slot, bi, pl.ds(0, 16)]
                    for k in range(1, NVREG):
                        t = t + accs[k] * vrows_v[slot, bi, pl.ds(k * 16, 16)]
                    s = jnp.sum(t) * inv_denom
                    return jnp.where(lane == bi + off, s, sv)

                svec = lax.fori_loop(0, CHUNK, elem, jnp.zeros((16,), jnp.float32))
                return svec

            issue(0, 0)

            def step(t, c):
                j0 = 2 * t
                issue(j0 + 1, 1)
                drain(0)
                svec0 = compute(j0, 0, 0)

                @pl.when(t < CPP // 2 - 1)
                def _():
                    issue(j0 + 2, 0)

                drain(1)
                svec1 = compute(j0 + 1, 1, CHUNK)
                packed = jnp.where(lane < CHUNK, svec0, svec1)
                score_v[pl.ds((pb + j0) * CHUNK, 16)] = packed
                return c

            lax.fori_loop(0, CPP // 2, step, 0)

        pltpu.sync_copy(score_v, out_hbm.at[pl.ds(wid * EPW, EPW)])

    return kfn(u_idx, v_idx, u_emb, v_emb)


def _tc_loss(scores):
    def body(s_ref, o_ref):
        x = s_ref[...]
        row = lax.broadcasted_iota(jnp.int32, x.shape, 0)
        y = jnp.where(row < x.shape[0] // 2, x, -x)
        o_ref[0, 0] = -jnp.sum(jax.nn.log_sigmoid(y)) / float(B)

    out = pl.pallas_call(
        body,
        out_shape=jax.ShapeDtypeStruct((1, 1), jnp.float32),
        out_specs=pl.BlockSpec(memory_space=pltpu.SMEM),
    )(scores.reshape(128, E // 128))
    return out[0, 0]


def kernel(pos_u, pos_v, neg_u, neg_v, u_emb, v_emb):
    u_idx = jnp.concatenate(
        [pos_u.reshape(-1), neg_u.reshape(-1)]
    ).astype(jnp.int32).reshape(NW, PHASES, UIPP)
    v_idx = jnp.concatenate([pos_v, neg_v]).astype(jnp.int32).reshape(NW, EPW)
    scores = _sc_scores(u_idx, v_idx, u_emb, v_emb)
    return _tc_loss(scores)


# speedup vs baseline: 1.0244x; 1.0244x over previous
import functools

import jax
import jax.numpy as jnp
from jax import lax
from jax.experimental import pallas as pl
from jax.experimental.pallas import tpu as pltpu
from jax.experimental.pallas import tpu_sc as plsc

NC = 2
NS = 16
NW = NC * NS

B = 16384
L = 20
D = 64
NVREG = D // 16
E = 2 * B
EPW = E // NW
CHUNK = 8
NCHUNK = EPW // CHUNK
UROWS = CHUNK * L
UGROUPS = UROWS // 16
PHASES = 2
CPP = NCHUNK // PHASES
UIPP = CPP * UROWS


def _sc_scores(u_idx, v_idx, u_emb, v_emb):
    mesh = plsc.VectorSubcoreMesh(
        core_axis_name="c", subcore_axis_name="s", num_cores=NC, num_subcores=NS
    )

    @functools.partial(
        pl.kernel,
        out_type=jax.ShapeDtypeStruct((E,), jnp.float32),
        mesh=mesh,
        compiler_params=pltpu.CompilerParams(needs_layout_passes=False),
        scratch_types=[
            pltpu.VMEM((UIPP,), jnp.int32),
            pltpu.VMEM((EPW + 16,), jnp.int32),
            pltpu.VMEM((2, UROWS, D), jnp.float32),
            pltpu.VMEM((2, CHUNK, D), jnp.float32),
            pltpu.VMEM((EPW,), jnp.float32),
            pltpu.SemaphoreType.DMA,
            pltpu.SemaphoreType.DMA,
            pltpu.SemaphoreType.DMA,
            pltpu.SemaphoreType.DMA,
        ],
    )
    def kfn(u_idx_hbm, v_idx_hbm, u_emb_hbm, v_emb_hbm, out_hbm,
            uidx_v, vidx_v, urows_v, vrows_v, score_v,
            usem0, usem1, vsem0, vsem1):
        wid = lax.axis_index("s") * NC + lax.axis_index("c")
        pltpu.sync_copy(v_idx_hbm.at[wid], vidx_v.at[pl.ds(0, EPW)])

        usems = (usem0, usem1)
        vsems = (vsem0, vsem1)
        inv_denom = 1.0 / float(L - 1)
        lane = lax.iota(jnp.int32, 16)

        for p in range(PHASES):
            pb = p * CPP
            pltpu.sync_copy(u_idx_hbm.at[wid, p], uidx_v)

            def issue(jj, slot):
                def ig(g, c):
                    iv = uidx_v[pl.ds((jj * UGROUPS + g) * 16, 16)]
                    idxs = [iv[k] for k in range(16)]
                    for k in range(16):
                        pltpu.async_copy(
                            u_emb_hbm.at[idxs[k]],
                            urows_v.at[slot, g * 16 + k],
                            usems[slot])
                    return c
                lax.fori_loop(0, UGROUPS, ig, 0)
                ivv = vidx_v[pl.ds((pb + jj) * CHUNK, 16)]
                idxs = [ivv[k] for k in range(CHUNK)]
                for k in range(CHUNK):
                    pltpu.async_copy(
                        v_emb_hbm.at[idxs[k]],
                        vrows_v.at[slot, k],
                        vsems[slot])

            def drain(slot):
                pltpu.make_async_copy(
                    u_emb_hbm.at[pl.ds(0, UROWS)], urows_v.at[slot],
                    usems[slot]).wait()
                pltpu.make_async_copy(
                    v_emb_hbm.at[pl.ds(0, CHUNK)], vrows_v.at[slot],
                    vsems[slot]).wait()

            def compute(jj, slot, off):
                def elem(bi, sv):
                    base = bi * L
                    accs = [urows_v[slot, base, pl.ds(k * 16, 16)]
                            for k in range(NVREG)]
                    for l in range(1, L):
                        for k in range(NVREG):
                            accs[k] = accs[k] + urows_v[slot, base + l,
                                                        pl.ds(k * 16, 16)]
                    t = accs[0] * vrows_v[slot, bi, pl.ds(0, 16)]
                    for k in range(1, NVREG):
                        t = t + accs[k] * vrows_v[slot, bi, pl.ds(k * 16, 16)]
                    s = jnp.sum(t) * inv_denom
                    return jnp.where(lane == bi + off, s, sv)

                svec = lax.fori_loop(0, CHUNK, elem, jnp.zeros((16,), jnp.float32))
                return svec

            issue(0, 0)

            def step(t, c):
                j0 = 2 * t
                issue(j0 + 1, 1)
                drain(0)
                svec0 = compute(j0, 0, 0)

                @pl.when(t < CPP // 2 - 1)
                def _():
                    issue(j0 + 2, 0)

                drain(1)
                svec1 = compute(j0 + 1, 1, CHUNK)
                packed = jnp.where(lane < CHUNK, svec0, svec1)
                score_v[pl.ds((pb + j0) * CHUNK, 16)] = packed
                return c

            lax.fori_loop(0, CPP // 2, step, 0)

        pltpu.sync_copy(score_v, out_hbm.at[pl.ds(wid * EPW, EPW)])

    return kfn(u_idx, v_idx, u_emb, v_emb)


def _tc_loss(scores):
    def body(s_ref, o_ref):
        x = s_ref[...]
        row = lax.broadcasted_iota(jnp.int32, x.shape, 0)
        y = jnp.where(row < x.shape[0] // 2, x, -x)
        o_ref[0, 0] = -jnp.sum(jax.nn.log_sigmoid(y)) / float(B)

    out = pl.pallas_call(
        body,
        out_shape=jax.ShapeDtypeStruct((1, 1), jnp.float32),
        out_specs=pl.BlockSpec(memory_space=pltpu.SMEM),
    )(scores.reshape(128, E // 128))
    return out[0, 0]


def kernel(pos_u, pos_v, neg_u, neg_v, u_emb, v_emb):
    u_idx = jnp.concatenate(
        [pos_u.reshape(-1), neg_u.reshape(-1)]
    ).astype(jnp.int32).reshape(NW, PHASES, UIPP)
    v_idx = jnp.concatenate([pos_v, neg_v]).astype(jnp.int32).reshape(NW, EPW)
    scores = _sc_scores(u_idx, v_idx, u_emb, v_emb)
    return _tc_loss(scores)
